# baseline (speedup 1.0000x reference)
import functools

import jax
import jax.numpy as jnp
from jax import lax
from jax.experimental import pallas as pl
from jax.experimental.pallas import tpu as pltpu
from jax.experimental.pallas import tpu_sc as plsc

_D = 1024
_CHUNK = 16
_NW = 32
_LANES = 16


def _sc_body(tok_hbm, gid_hbm, table_hbm, out_hbm,
             idx_v, vec_v, in0, in1, out0, out1,
             sem_vec, si0, si1, so0, so1):
    nc = 2
    c = lax.axis_index("c")
    s = lax.axis_index("s")
    wid = s * nc + c
    rows = tok_hbm.shape[0]
    rpw = rows // _NW
    base = wid * rpw
    nchunk = rpw // _CHUNK

    pltpu.sync_copy(gid_hbm, idx_v)
    pltpu.async_copy(table_hbm.at[idx_v], vec_v, sem_vec).wait()

    in_bufs = (in0, in1)
    out_bufs = (out0, out1)
    in_sems = (si0, si1)
    out_sems = (so0, so1)

    for b in range(2):
        pltpu.async_copy(
            tok_hbm.at[pl.ds(base + b * _CHUNK, _CHUNK)], in_bufs[b], in_sems[b])

    def _add_rows(ib, ob):
        def row(r, carry):
            for j in range(_D // _LANES):
                sl = pl.ds(j * _LANES, _LANES)
                ob[r, sl] = ib[r, sl] + vec_v[0, sl]
            return carry
        lax.fori_loop(0, _CHUNK, row, 0)

    def _step(g, b):
        ib, ob = in_bufs[b], out_bufs[b]
        pltpu.make_async_copy(
            tok_hbm.at[pl.ds(0, _CHUNK)], ib, in_sems[b]).wait()
        @pl.when(g >= 2)
        def _():
            pltpu.make_async_copy(
                ob, out_hbm.at[pl.ds(0, _CHUNK)], out_sems[b]).wait()
        _add_rows(ib, ob)
        @pl.when(g + 2 < nchunk)
        def _():
            pltpu.async_copy(
                tok_hbm.at[pl.ds(base + (g + 2) * _CHUNK, _CHUNK)], ib, in_sems[b])
        pltpu.async_copy(
            ob, out_hbm.at[pl.ds(base + g * _CHUNK, _CHUNK)], out_sems[b])

    def _outer(i, carry):
        for b in range(2):
            _step(i * 2 + b, b)
        return carry

    lax.fori_loop(0, nchunk // 2, _outer, 0)

    for b in range(2):
        pltpu.make_async_copy(
            out_bufs[b], out_hbm.at[pl.ds(0, _CHUNK)], out_sems[b]).wait()


def kernel(tokens, group_id, group_id_vecs):
    b, s, d = tokens.shape
    rows = b * s
    tok2d = tokens.reshape(rows, d)
    gid = jnp.asarray(group_id, jnp.int32).reshape((1,))

    sc_add = pl.kernel(
        _sc_body,
        out_type=jax.ShapeDtypeStruct((rows, d), tokens.dtype),
        mesh=plsc.VectorSubcoreMesh(core_axis_name="c", subcore_axis_name="s"),
        scratch_types=[
            pltpu.VMEM((1,), jnp.int32),
            pltpu.VMEM((1, d), jnp.float32),
            pltpu.VMEM((_CHUNK, d), jnp.float32),
            pltpu.VMEM((_CHUNK, d), jnp.float32),
            pltpu.VMEM((_CHUNK, d), jnp.float32),
            pltpu.VMEM((_CHUNK, d), jnp.float32),
            pltpu.SemaphoreType.DMA,
            pltpu.SemaphoreType.DMA,
            pltpu.SemaphoreType.DMA,
            pltpu.SemaphoreType.DMA,
            pltpu.SemaphoreType.DMA,
        ],
    )
    out = sc_add(tok2d, gid, group_id_vecs)
    return out.reshape(b, s, d)
